# baseline (speedup 1.0000x reference)
import functools

import jax
import jax.numpy as jnp
from jax import lax
from jax.experimental import pallas as pl
from jax.experimental.pallas import tpu as pltpu
from jax.experimental.pallas import tpu_sc as plsc

_N = 10000
_E = 320000
_D = 128
_NC = 2
_NS = 16
_NW = _NC * _NS
_EPT = _E // _NW
_CH = 125
_EB = _E // _CH
_CPT = _EPT // _CH
_NP = 10240
_RPT = _NP // _NS

_mesh = plsc.VectorSubcoreMesh(core_axis_name="c", subcore_axis_name="s")


def _fill2d(ref, rows, val):
    v = jnp.full((16,), val, jnp.float32)

    def _o(i, _):
        def _i(j, _):
            ref[i, pl.ds(j * 16, 16)] = v
            return 0

        lax.fori_loop(0, _D // 16, _i, 0)
        return 0

    lax.fori_loop(0, rows, _o, 0)


def _zero_stripe(src_v, acc_sh, s):
    sb = s * _RPT

    def _z(k, _):
        pltpu.sync_copy(src_v.at[pl.ds(0, 80)],
                        acc_sh.at[pl.ds(sb + k * 80, 80)])
        return 0

    lax.fori_loop(0, _RPT // 80, _z, 0)


def _stripe_out(acc_sh, out_hbm, c, s):
    sb = s * _RPT
    pltpu.sync_copy(acc_sh.at[pl.ds(sb, _RPT)], out_hbm.at[c, pl.ds(sb, _RPT)])


@functools.partial(
    pl.kernel,
    out_type=jax.ShapeDtypeStruct((_NC, _NP, _D), jnp.float32),
    mesh=_mesh,
    scratch_types=[
        pltpu.VMEM((_CPT, _CH), jnp.int32),
        pltpu.VMEM((_CH, _D), jnp.float32),
        pltpu.VMEM_SHARED((_NP, _D), jnp.float32),
        pltpu.SemaphoreType.DMA,
    ],
)
def _deg_kernel(dstb_hbm, out_hbm, didx_v, ones_v, deg_sh, sem):
    c = lax.axis_index("c")
    s = lax.axis_index("s")
    wid = c * _NS + s
    rbase = wid * _CPT

    pltpu.sync_copy(dstb_hbm.at[pl.ds(rbase, _CPT)], didx_v)
    _fill2d(ones_v, _CH, 0.0)
    _zero_stripe(ones_v, deg_sh, s)
    _fill2d(ones_v, _CH, 1.0)
    plsc.subcore_barrier()

    def _fire(i, _):
        pltpu.async_copy(ones_v, deg_sh.at[didx_v.at[i]], sem, add=True)
        return 0

    lax.fori_loop(0, _CPT, _fire, 0)

    def _drain(i, _):
        pltpu.make_async_copy(ones_v, deg_sh.at[didx_v.at[0]], sem).wait()
        return 0

    lax.fori_loop(0, _CPT, _drain, 0)
    plsc.subcore_barrier()
    _stripe_out(deg_sh, out_hbm, c, s)


_CA = 80
_CPA = _EPT // _CA


@functools.partial(
    pl.kernel,
    out_type=jax.ShapeDtypeStruct((_NC, _NP, _D), jnp.float32),
    mesh=_mesh,
    scratch_types=[
        tuple(pltpu.VMEM((_CA,), jnp.int32) for _ in range(8)),
        tuple(pltpu.VMEM((_CA,), jnp.int32) for _ in range(8)),
        tuple(pltpu.VMEM((_CA, _D), jnp.float32) for _ in range(4)),
        pltpu.VMEM_SHARED((_NP, _D), jnp.float32),
        tuple(pltpu.SemaphoreType.DMA for _ in range(4)),
        tuple(pltpu.SemaphoreType.DMA for _ in range(4)),
        tuple(pltpu.SemaphoreType.DMA for _ in range(8)),
    ],
)
def _agg_kernel(hp_hbm, src_hbm, dst_hbm, out_hbm, sidx, didx, rows,
                acc_sh, gsem, ssem, isem):
    c = lax.axis_index("c")
    s = lax.axis_index("s")
    wid = c * _NS + s
    ebase = wid * _EPT

    def _ldidx(k):
        st = k % 8 if isinstance(k, int) else None
        pltpu.async_copy(src_hbm.at[pl.ds(ebase + k * _CA, _CA)], sidx[st],
                         isem[st])
        pltpu.async_copy(dst_hbm.at[pl.ds(ebase + k * _CA, _CA)], didx[st],
                         isem[st])

    def _ldidx_t(kt, st):
        pltpu.async_copy(src_hbm.at[pl.ds(ebase + kt * _CA, _CA)], sidx[st],
                         isem[st])
        pltpu.async_copy(dst_hbm.at[pl.ds(ebase + kt * _CA, _CA)], didx[st],
                         isem[st])

    def _widx(st):
        pltpu.make_async_copy(src_hbm.at[pl.ds(ebase, _CA)], sidx[st],
                              isem[st]).wait()
        pltpu.make_async_copy(dst_hbm.at[pl.ds(ebase, _CA)], didx[st],
                              isem[st]).wait()

    def _g(st, b):
        pltpu.async_copy(hp_hbm.at[sidx[st]], rows[b], gsem[b])

    def _wg(b):
        pltpu.make_async_copy(hp_hbm.at[sidx[0]], rows[b], gsem[b]).wait()

    def _sc(st, b):
        pltpu.async_copy(rows[b], acc_sh.at[didx[st]], ssem[b], add=True)

    def _ws(b):
        pltpu.make_async_copy(rows[b], acc_sh.at[didx[0]], ssem[b]).wait()

    _fill2d(rows[0], 80, 0.0)
    _zero_stripe(rows[0], acc_sh, s)
    plsc.subcore_barrier()

    for k0 in range(6):
        _ldidx(k0)
    _widx(0)
    _g(0, 0)
    _widx(1)
    _g(1, 1)

    def _step(kt, jj, in_loop):
        b = jj % 4
        _wg(b)
        _sc(jj, b)
        if in_loop:
            @pl.when(kt >= 2)
            def _():
                _ws((jj - 2) % 4)
        else:
            _ws((jj - 2) % 4)
        if in_loop:
            @pl.when(kt + 6 < _CPA)
            def _():
                _ldidx_t(kt + 6, (jj + 6) % 8)
            _widx((jj + 2) % 8)
            _g((jj + 2) % 8, (jj + 2) % 4)

    def _body(q, _):
        for jj in range(8):
            _step(8 * q + jj, jj, True)
        return 0

    lax.fori_loop(0, _CPA // 8, _body, 0)
    for k in range(120, _CPA):
        jj = k % 8
        b = jj % 4
        _wg(b)
        _sc(jj, b)
        _ws((jj - 2) % 4)
        if k + 2 < _CPA:
            _widx((jj + 2) % 8)
            _g((jj + 2) % 8, (jj + 2) % 4)
    _ws(123 % 4)
    _ws(124 % 4)
    plsc.subcore_barrier()
    _stripe_out(acc_sh, out_hbm, c, s)


_BM = 1000


def _dis(deg_ref):
    return lax.rsqrt(deg_ref[0] + deg_ref[1] + 1.0)


def _mm1_body(deg_ref, x_ref, w_ref, o_ref):
    o_ref[...] = jnp.dot(x_ref[...], w_ref[...],
                         preferred_element_type=jnp.float32) * _dis(deg_ref)


def _mid_body(deg_ref, p_ref, hp_ref, b_ref, w_ref, o_ref):
    dis = _dis(deg_ref)
    z = (p_ref[0] + p_ref[1] + hp_ref[...]) * dis + b_ref[...]
    z = jnp.maximum(z, 0.0)
    o_ref[...] = jnp.dot(z, w_ref[...],
                         preferred_element_type=jnp.float32) * dis


def _fin_body(deg_ref, p_ref, hp_ref, b2_ref, wp1_ref, bp1_ref, wp2_ref,
              bp2_ref, o_ref):
    dis = _dis(deg_ref)
    z = (p_ref[0] + p_ref[1] + hp_ref[...]) * dis + b2_ref[...]
    z = jnp.maximum(z, 0.0)
    y = jnp.dot(z, wp1_ref[...], preferred_element_type=jnp.float32)
    y = y + bp1_ref[...]
    y = jnp.dot(y, wp2_ref[...], preferred_element_type=jnp.float32)
    y = y + bp2_ref[...]
    m = jnp.max(y, axis=1, keepdims=True)
    e = jnp.exp(y - m)
    o_ref[...] = y - m - jnp.log(jnp.sum(e, axis=1, keepdims=True))


_deg_spec = pl.BlockSpec((_NC, _BM, 1), lambda i: (0, i, 0))
_row_spec = pl.BlockSpec((_BM, _D), lambda i: (i, 0))
_p_spec = pl.BlockSpec((_NC, _BM, _D), lambda i: (0, i, 0))
_w_spec = pl.BlockSpec((_D, _D), lambda i: (0, 0))
_b_spec = pl.BlockSpec((1, _D), lambda i: (0, 0))
_grid = (_N // _BM,)
_row_out = jax.ShapeDtypeStruct((_N, _D), jnp.float32)

_mm1 = pl.pallas_call(
    _mm1_body, grid=_grid,
    in_specs=[_deg_spec, _row_spec, _w_spec],
    out_specs=_row_spec, out_shape=_row_out)

_mid = pl.pallas_call(
    _mid_body, grid=_grid,
    in_specs=[_deg_spec, _p_spec, _row_spec, _b_spec, _w_spec],
    out_specs=_row_spec, out_shape=_row_out)

_fin = pl.pallas_call(
    _fin_body, grid=_grid,
    in_specs=[_deg_spec, _p_spec, _row_spec, _b_spec, _w_spec, _b_spec,
              _w_spec, _b_spec],
    out_specs=_row_spec, out_shape=_row_out)


def kernel(x, edge_index, W1, b1, W2, b2, Wp1, bp1, Wp2, bp2):
    src = edge_index[0]
    dst = edge_index[1]
    dstb = edge_index.reshape(2, _EB, _CH)[1]
    degp = _deg_kernel(dstb)[:, :, :1]
    b1r = b1.reshape(1, _D)
    b2r = b2.reshape(1, _D)
    bp1r = bp1.reshape(1, _D)
    bp2r = bp2.reshape(1, _D)

    hp1 = _mm1(degp, x, W1)
    p1 = _agg_kernel(hp1, src, dst)
    hp2 = _mid(degp, p1, hp1, b1r, W2)
    p2 = _agg_kernel(hp2, src, dst)
    return _fin(degp, p2, hp2, b2r, Wp1, bp1r, Wp2, bp2r)


# speedup vs baseline: 1.0480x; 1.0480x over previous
import functools

import jax
import jax.numpy as jnp
from jax import lax
from jax.experimental import pallas as pl
from jax.experimental.pallas import tpu as pltpu
from jax.experimental.pallas import tpu_sc as plsc

_N = 10000
_E = 320000
_D = 128
_NC = 2
_NS = 16
_NW = _NC * _NS
_EPT = _E // _NW
_CH = 125
_EB = _E // _CH
_CPT = _EPT // _CH
_NP = 10240
_RPT = _NP // _NS

_mesh = plsc.VectorSubcoreMesh(core_axis_name="c", subcore_axis_name="s")


def _fill2d(ref, rows, val):
    v = jnp.full((16,), val, jnp.float32)

    def _o(i, _):
        def _i(j, _):
            ref[i, pl.ds(j * 16, 16)] = v
            return 0

        lax.fori_loop(0, _D // 16, _i, 0)
        return 0

    lax.fori_loop(0, rows, _o, 0)


def _zero_stripe(src_v, acc_sh, s):
    sb = s * _RPT

    def _z(k, _):
        pltpu.sync_copy(src_v.at[pl.ds(0, 80)],
                        acc_sh.at[pl.ds(sb + k * 80, 80)])
        return 0

    lax.fori_loop(0, _RPT // 80, _z, 0)


def _stripe_out(acc_sh, out_hbm, c, s):
    sb = s * _RPT
    pltpu.sync_copy(acc_sh.at[pl.ds(sb, _RPT)], out_hbm.at[c, pl.ds(sb, _RPT)])


@functools.partial(
    pl.kernel,
    out_type=jax.ShapeDtypeStruct((_NC, _NP, _D), jnp.float32),
    mesh=_mesh,
    scratch_types=[
        pltpu.VMEM((_CPT, _CH), jnp.int32),
        pltpu.VMEM((_CH, _D), jnp.float32),
        pltpu.VMEM_SHARED((_NP, _D), jnp.float32),
        pltpu.SemaphoreType.DMA,
    ],
)
def _deg_kernel(dstb_hbm, out_hbm, didx_v, ones_v, deg_sh, sem):
    c = lax.axis_index("c")
    s = lax.axis_index("s")
    wid = c * _NS + s
    rbase = wid * _CPT

    pltpu.sync_copy(dstb_hbm.at[pl.ds(rbase, _CPT)], didx_v)
    _fill2d(ones_v, _CH, 0.0)
    _zero_stripe(ones_v, deg_sh, s)
    _fill2d(ones_v, _CH, 1.0)
    plsc.subcore_barrier()

    def _fire(i, _):
        pltpu.async_copy(ones_v, deg_sh.at[didx_v.at[i]], sem, add=True)
        return 0

    lax.fori_loop(0, _CPT, _fire, 0)

    def _drain(i, _):
        pltpu.make_async_copy(ones_v, deg_sh.at[didx_v.at[0]], sem).wait()
        return 0

    lax.fori_loop(0, _CPT, _drain, 0)
    plsc.subcore_barrier()
    _stripe_out(deg_sh, out_hbm, c, s)


_NPAIR = _CPT // 16


@functools.partial(
    pl.kernel,
    out_type=jax.ShapeDtypeStruct((_NC, _NP, _D), jnp.float32),
    mesh=_mesh,
    scratch_types=[
        pltpu.VMEM((8, _CH), jnp.int32),
        pltpu.VMEM((8, _CH), jnp.int32),
        pltpu.VMEM((8, _CH), jnp.int32),
        pltpu.VMEM((8, _CH), jnp.int32),
        pltpu.VMEM((_CH, _D), jnp.float32),
        pltpu.VMEM((_CH, _D), jnp.float32),
        pltpu.VMEM_SHARED((_NP, _D), jnp.float32),
        pltpu.SemaphoreType.DMA,
        pltpu.SemaphoreType.DMA,
        pltpu.SemaphoreType.DMA,
        pltpu.SemaphoreType.DMA,
        pltpu.SemaphoreType.DMA,
        pltpu.SemaphoreType.DMA,
    ],
)
def _agg_kernel(hp_hbm, srcb_hbm, dstb_hbm, out_hbm, s0, s1, d0, d1, ra, rb,
                acc_sh, ga, gb, sa, sb, i0, i1):
    c = lax.axis_index("c")
    s = lax.axis_index("s")
    wid = c * _NS + s
    rbase = wid * _CPT

    sidx = (s0, s1)
    didx = (d0, d1)
    rows = (ra, rb)
    gsem = (ga, gb)
    ssem = (sa, sb)
    isem = (i0, i1)

    def _ldidx(o, st):
        pltpu.async_copy(srcb_hbm.at[pl.ds(rbase + o * 8, 8)], sidx[st],
                         isem[st])
        pltpu.async_copy(dstb_hbm.at[pl.ds(rbase + o * 8, 8)], didx[st],
                         isem[st])

    def _widx(st):
        pltpu.make_async_copy(srcb_hbm.at[pl.ds(rbase, 8)], sidx[st],
                              isem[st]).wait()
        pltpu.make_async_copy(dstb_hbm.at[pl.ds(rbase, 8)], didx[st],
                              isem[st]).wait()

    def _g(st, row, b):
        pltpu.async_copy(hp_hbm.at[sidx[st].at[row]], rows[b], gsem[b])

    def _wg(b):
        pltpu.make_async_copy(hp_hbm.at[sidx[0].at[0]], rows[b],
                              gsem[b]).wait()

    def _sc(st, row, b):
        pltpu.async_copy(rows[b], acc_sh.at[didx[st].at[row]], ssem[b],
                         add=True)

    def _ws(b):
        pltpu.make_async_copy(rows[b], acc_sh.at[didx[0].at[0]],
                              ssem[b]).wait()

    _fill2d(ra, 80, 0.0)
    _zero_stripe(ra, acc_sh, s)
    plsc.subcore_barrier()

    _ldidx(0, 0)
    _widx(0)
    _g(0, 0, 0)
    _g(0, 1, 1)

    def _body(q, _):
        for jj in range(16):
            ci = 16 * q + jj
            b = jj % 2
            _wg(b)
            st, row = (0, jj) if jj < 8 else (1, jj - 8)
            _sc(st, row, b)
            if jj == 1:
                _ldidx(2 * q + 1, 1)
            if jj == 9:
                @pl.when(q < _NPAIR - 1)
                def _():
                    _ldidx(2 * q + 2, 0)
            if jj == 6:
                _widx(1)
            _ws(b)
            if jj < 6:
                gst, grow = 0, jj + 2
            elif jj < 14:
                gst, grow = 1, jj - 6
            else:
                gst, grow = 0, jj - 14
            if jj >= 14:
                @pl.when(q < _NPAIR - 1)
                def _():
                    if jj == 14:
                        _widx(0)
                    _g(gst, grow, b)
            else:
                _g(gst, grow, b)
        return 0

    lax.fori_loop(0, _NPAIR, _body, 0)
    plsc.subcore_barrier()
    _stripe_out(acc_sh, out_hbm, c, s)


_BM = 1000


def _dis(deg_ref):
    return lax.rsqrt(deg_ref[0] + deg_ref[1] + 1.0)


def _mm1_body(x_ref, w_ref, o_ref):
    o_ref[...] = jnp.dot(x_ref[...], w_ref[...],
                         preferred_element_type=jnp.float32)


def _scale_body(deg_ref, h_ref, o_ref):
    o_ref[...] = h_ref[...] * _dis(deg_ref)


def _mid_body(deg_ref, p_ref, hp_ref, b_ref, w_ref, o_ref):
    dis = _dis(deg_ref)
    z = (p_ref[0] + p_ref[1] + hp_ref[...]) * dis + b_ref[...]
    z = jnp.maximum(z, 0.0)
    o_ref[...] = jnp.dot(z, w_ref[...],
                         preferred_element_type=jnp.float32) * dis


def _fin_body(deg_ref, p_ref, hp_ref, b2_ref, wp1_ref, bp1_ref, wp2_ref,
              bp2_ref, o_ref):
    dis = _dis(deg_ref)
    z = (p_ref[0] + p_ref[1] + hp_ref[...]) * dis + b2_ref[...]
    z = jnp.maximum(z, 0.0)
    y = jnp.dot(z, wp1_ref[...], preferred_element_type=jnp.float32)
    y = y + bp1_ref[...]
    y = jnp.dot(y, wp2_ref[...], preferred_element_type=jnp.float32)
    y = y + bp2_ref[...]
    m = jnp.max(y, axis=1, keepdims=True)
    e = jnp.exp(y - m)
    o_ref[...] = y - m - jnp.log(jnp.sum(e, axis=1, keepdims=True))


_deg_spec = pl.BlockSpec((_NC, _BM, 1), lambda i: (0, i, 0))
_row_spec = pl.BlockSpec((_BM, _D), lambda i: (i, 0))
_p_spec = pl.BlockSpec((_NC, _BM, _D), lambda i: (0, i, 0))
_w_spec = pl.BlockSpec((_D, _D), lambda i: (0, 0))
_b_spec = pl.BlockSpec((1, _D), lambda i: (0, 0))
_grid = (_N // _BM,)
_row_out = jax.ShapeDtypeStruct((_N, _D), jnp.float32)

_mm1 = pl.pallas_call(
    _mm1_body, grid=_grid,
    in_specs=[_row_spec, _w_spec],
    out_specs=_row_spec, out_shape=_row_out)

_scale = pl.pallas_call(
    _scale_body, grid=_grid,
    in_specs=[_deg_spec, _row_spec],
    out_specs=_row_spec, out_shape=_row_out)

_mid = pl.pallas_call(
    _mid_body, grid=_grid,
    in_specs=[_deg_spec, _p_spec, _row_spec, _b_spec, _w_spec],
    out_specs=_row_spec, out_shape=_row_out)

_fin = pl.pallas_call(
    _fin_body, grid=_grid,
    in_specs=[_deg_spec, _p_spec, _row_spec, _b_spec, _w_spec, _b_spec,
              _w_spec, _b_spec],
    out_specs=_row_spec, out_shape=_row_out)


def kernel(x, edge_index, W1, b1, W2, b2, Wp1, bp1, Wp2, bp2):
    e3 = edge_index.reshape(2, _EB, _CH)
    srcb = e3[0]
    dstb = e3[1]
    degp = _deg_kernel(dstb)[:, :, :1]
    b1r = b1.reshape(1, _D)
    b2r = b2.reshape(1, _D)
    bp1r = bp1.reshape(1, _D)
    bp2r = bp2.reshape(1, _D)

    h1 = _mm1(x, W1)
    hp1 = _scale(degp, h1)
    p1 = _agg_kernel(hp1, srcb, dstb)
    hp2 = _mid(degp, p1, hp1, b1r, W2)
    p2 = _agg_kernel(hp2, srcb, dstb)
    return _fin(degp, p2, hp2, b2r, Wp1, bp1r, Wp2, bp2r)
